# baseline (speedup 1.0000x reference)
import functools

import jax
import jax.numpy as jnp
import numpy as np
from jax import lax
from jax.experimental import pallas as pl
from jax.experimental.pallas import tpu as pltpu
from jax.experimental.pallas import tpu_sc as plsc

_POOLING_RATE = 4
_K = 16

_PERM_CACHE = {}


def _sample_indices(n, p):
    if n not in _PERM_CACHE:
        with jax.ensure_compile_time_eval():
            perm = jax.random.permutation(jax.random.key(42), n)
        _PERM_CACHE[n] = np.asarray(perm).astype(np.int32)
    return _PERM_CACHE[n][:p]


_BR = 256


_CW = 128
_T = 4


def _knn_body(vs_ref, vt_ref, out_ref):
    vsb = vs_ref[0]
    vt = vt_ref[0]
    n = vt.shape[1]
    nch = n // _CW
    inner = lax.dot_general(vsb, vt, (((1,), (0,)), ((), ())),
                            preferred_element_type=jnp.float32)
    q = jnp.sum(vt * vt, axis=0)
    qs = jnp.sum(vsb * vsb, axis=1)
    d = (-2.0 * inner + q[None, :]) + qs[:, None]
    off = pl.program_id(0) * n
    big = jnp.float32(1e30)
    bigi = jnp.int32(1 << 30)

    dl = [d[:, a * _CW:(a + 1) * _CW] for a in range(nch)]
    li = lax.broadcasted_iota(jnp.int32, dl[0].shape, 1)
    vals, colx = [], []
    for _ in range(_T):
        m = dl[0]
        for a in range(1, nch):
            m = jnp.minimum(m, dl[a])
        eqs = [dl[a] == m for a in range(nch)]
        pos = jnp.full(m.shape, nch - 1, jnp.int32)
        for a in range(nch - 2, -1, -1):
            pos = jnp.where(eqs[a], a, pos)
        vals.append(m)
        colx.append(pos * _CW + li)
        dl = [jnp.where(eqs[a], big, dl[a]) for a in range(nch)]

    for k in range(_K + 1):
        m = vals[0]
        for t in range(1, _T):
            m = jnp.minimum(m, vals[t])
        mrow = jnp.min(m, axis=1)
        eqs = [vals[t] == mrow[:, None] for t in range(_T)]
        cc = jnp.where(eqs[0], colx[0], bigi)
        for t in range(1, _T):
            cc = jnp.minimum(cc, jnp.where(eqs[t], colx[t], bigi))
        idx = jnp.min(cc, axis=1)
        vals = [jnp.where(eqs[t], big, vals[t]) for t in range(_T)]
        if k > 0:
            out_ref[0, k - 1, :] = idx + off


def _knn_indices(vs, vt):
    bs, p, _ = vs.shape
    n = vt.shape[2]
    return pl.pallas_call(
        _knn_body,
        grid=(bs, p // _BR),
        in_specs=[
            pl.BlockSpec((1, _BR, 3), lambda b, r: (b, r, 0)),
            pl.BlockSpec((1, 3, n), lambda b, r: (b, 0, 0)),
        ],
        out_specs=pl.BlockSpec((1, _K, _BR), lambda b, r: (b, 0, r)),
        out_shape=jax.ShapeDtypeStruct((bs, _K, p), jnp.int32),
    )(vs, vt)


_CP = 16


def _make_pool_kernel(rows_total, d, pw, nbatch, p):
    info = plsc.get_sparse_core_info()
    nc, ns = info.num_cores, info.num_subcores
    steps = pw // _CP
    wpb = p // pw

    mesh = plsc.VectorSubcoreMesh(core_axis_name="c", subcore_axis_name="s")

    assert steps % 2 == 0

    @functools.partial(
        pl.kernel,
        mesh=mesh,
        out_type=jax.ShapeDtypeStruct((rows_total, d), jnp.bfloat16),
        scratch_types=[
            pltpu.VMEM((pw * _K,), jnp.int32),
            pltpu.VMEM((2, _CP * _K, d), jnp.bfloat16),
            pltpu.VMEM((2, _CP, d), jnp.bfloat16),
            pltpu.SemaphoreType.DMA,
            pltpu.SemaphoreType.DMA,
        ],
        compiler_params=pltpu.CompilerParams(use_tc_tiling_on_sc=False),
    )
    def pool_kernel(fm_hbm, nbr_hbm, out_hbm, idx_v, gbuf, obuf, semA, semB):
        wid = lax.axis_index("s") * nc + lax.axis_index("c")
        gbase = wid * pw
        pltpu.sync_copy(nbr_hbm.at[pl.ds(gbase * _K, pw * _K)], idx_v)

        def issue(s, buf, sem):
            return pltpu.async_copy(
                fm_hbm.at[idx_v.at[pl.ds(s * (_CP * _K), _CP * _K)]],
                gbuf.at[buf], sem)

        def drain(s, buf, sem):
            pltpu.make_async_copy(
                fm_hbm.at[idx_v.at[pl.ds(s * (_CP * _K), _CP * _K)]],
                gbuf.at[buf], sem).wait()

        def reduce_store(s, buf):
            for pp in range(_CP):
                base = pp * _K
                for j in range(d // 32):
                    sl = pl.ds(j * 32, 32)
                    acc = gbuf[buf, base, sl]
                    for k in range(1, _K):
                        acc = jnp.maximum(acc, gbuf[buf, base + k, sl])
                    obuf[buf, pp, sl] = acc
            pltpu.sync_copy(obuf.at[buf],
                            out_hbm.at[pl.ds(gbase + s * _CP, _CP)])

        issue(0, 0, semA)

        def step(t, carry):
            s0 = 2 * t
            issue(s0 + 1, 1, semB)
            drain(s0, 0, semA)
            reduce_store(s0, 0)

            @pl.when(s0 + 2 < steps)
            def _():
                issue(s0 + 2, 0, semA)

            drain(s0 + 1, 1, semB)
            reduce_store(s0 + 1, 1)
            return carry

        lax.fori_loop(0, steps // 2, step, 0)

    return pool_kernel


def _run_half(vertices, feature_map, sample_idx):
    bs, n, _ = vertices.shape
    c = feature_map.shape[1]
    r = feature_map.shape[-1]
    p = n // _POOLING_RATE
    d = c * r

    vt = jnp.transpose(vertices, (0, 2, 1))
    vs = jnp.take(vertices, sample_idx, axis=1)
    nbr = _knn_indices(vs, vt)

    fm_t = jnp.transpose(feature_map, (0, 2, 1, 3)).astype(
        jnp.bfloat16).reshape(bs * n, d)
    nbr_flat = jnp.transpose(nbr, (0, 2, 1)).reshape(-1)

    nw = 32
    pw = (bs * p) // nw
    pool = _make_pool_kernel(bs * p, d, pw, bs, p)
    pooled = pool(fm_t, nbr_flat)

    feature_map_pool = pooled.astype(jnp.float32).reshape(
        bs, p, c, r).transpose(0, 2, 1, 3)
    return (vs, feature_map_pool)


def kernel(vertices, feature_map):
    bs, n, _ = vertices.shape
    p = n // _POOLING_RATE
    sample_idx = _sample_indices(n, p)
    nsplit = 4
    hb = bs // nsplit
    parts = [
        _run_half(vertices[h * hb:(h + 1) * hb],
                  feature_map[h * hb:(h + 1) * hb], sample_idx)
        for h in range(nsplit)
    ]
    return (jnp.concatenate([v for v, _ in parts], axis=0),
            jnp.concatenate([f for _, f in parts], axis=0))


# speedup vs baseline: 1.1105x; 1.1105x over previous
import functools

import jax
import jax.numpy as jnp
import numpy as np
from jax import lax
from jax.experimental import pallas as pl
from jax.experimental.pallas import tpu as pltpu
from jax.experimental.pallas import tpu_sc as plsc

_POOLING_RATE = 4
_K = 16

_PERM_CACHE = {}


def _sample_indices(n, p):
    if n not in _PERM_CACHE:
        with jax.ensure_compile_time_eval():
            perm = jax.random.permutation(jax.random.key(42), n)
        _PERM_CACHE[n] = np.asarray(perm).astype(np.int32)
    return _PERM_CACHE[n][:p]


_BR = 512


_CW = 128
_T = 4


def _knn_body(vs_ref, vt_ref, out_ref):
    vsb = vs_ref[0]
    vt = vt_ref[0]
    n = vt.shape[1]
    nch = n // _CW
    inner = lax.dot_general(vsb, vt, (((1,), (0,)), ((), ())),
                            preferred_element_type=jnp.float32)
    q = jnp.sum(vt * vt, axis=0)
    qs = jnp.sum(vsb * vsb, axis=1)
    d = (-2.0 * inner + q[None, :]) + qs[:, None]
    off = pl.program_id(0) * n
    big = jnp.float32(1e30)
    bigi = jnp.int32(1 << 30)

    dl = [d[:, a * _CW:(a + 1) * _CW] for a in range(nch)]
    li = lax.broadcasted_iota(jnp.int32, dl[0].shape, 1)
    vals, colx = [], []
    for _ in range(_T):
        m = dl[0]
        for a in range(1, nch):
            m = jnp.minimum(m, dl[a])
        eqs = [dl[a] == m for a in range(nch)]
        pos = jnp.full(m.shape, nch - 1, jnp.int32)
        for a in range(nch - 2, -1, -1):
            pos = jnp.where(eqs[a], a, pos)
        vals.append(m)
        colx.append(pos * _CW + li)
        dl = [jnp.where(eqs[a], big, dl[a]) for a in range(nch)]

    for k in range(_K + 1):
        m = vals[0]
        for t in range(1, _T):
            m = jnp.minimum(m, vals[t])
        mrow = jnp.min(m, axis=1)
        eqs = [vals[t] == mrow[:, None] for t in range(_T)]
        cc = jnp.where(eqs[0], colx[0], bigi)
        for t in range(1, _T):
            cc = jnp.minimum(cc, jnp.where(eqs[t], colx[t], bigi))
        idx = jnp.min(cc, axis=1)
        vals = [jnp.where(eqs[t], big, vals[t]) for t in range(_T)]
        if k > 0:
            out_ref[0, k - 1, :] = idx + off


def _knn_indices(vs, vt):
    bs, p, _ = vs.shape
    n = vt.shape[2]
    return pl.pallas_call(
        _knn_body,
        grid=(bs, p // _BR),
        in_specs=[
            pl.BlockSpec((1, _BR, 3), lambda b, r: (b, r, 0)),
            pl.BlockSpec((1, 3, n), lambda b, r: (b, 0, 0)),
        ],
        out_specs=pl.BlockSpec((1, _K, _BR), lambda b, r: (b, 0, r)),
        out_shape=jax.ShapeDtypeStruct((bs, _K, p), jnp.int32),
    )(vs, vt)


_CP = 16


def _make_pool_kernel(rows_total, d, pw, nbatch, p):
    info = plsc.get_sparse_core_info()
    nc, ns = info.num_cores, info.num_subcores
    steps = pw // _CP
    wpb = p // pw

    mesh = plsc.VectorSubcoreMesh(core_axis_name="c", subcore_axis_name="s")

    assert steps % 2 == 0

    @functools.partial(
        pl.kernel,
        mesh=mesh,
        out_type=jax.ShapeDtypeStruct((rows_total, d), jnp.bfloat16),
        scratch_types=[
            pltpu.VMEM((pw * _K,), jnp.int32),
            pltpu.VMEM((2, _CP * _K, d), jnp.bfloat16),
            pltpu.VMEM((2, _CP, d), jnp.bfloat16),
            pltpu.SemaphoreType.DMA,
            pltpu.SemaphoreType.DMA,
        ],
        compiler_params=pltpu.CompilerParams(use_tc_tiling_on_sc=False),
    )
    def pool_kernel(fm_hbm, nbr_hbm, out_hbm, idx_v, gbuf, obuf, semA, semB):
        wid = lax.axis_index("s") * nc + lax.axis_index("c")
        gbase = wid * pw
        pltpu.sync_copy(nbr_hbm.at[pl.ds(gbase * _K, pw * _K)], idx_v)

        def issue(s, buf, sem):
            return pltpu.async_copy(
                fm_hbm.at[idx_v.at[pl.ds(s * (_CP * _K), _CP * _K)]],
                gbuf.at[buf], sem)

        def drain(s, buf, sem):
            pltpu.make_async_copy(
                fm_hbm.at[idx_v.at[pl.ds(s * (_CP * _K), _CP * _K)]],
                gbuf.at[buf], sem).wait()

        def reduce_store(s, buf):
            for pp in range(_CP):
                base = pp * _K
                for j in range(d // 32):
                    sl = pl.ds(j * 32, 32)
                    acc = gbuf[buf, base, sl]
                    for k in range(1, _K):
                        acc = jnp.maximum(acc, gbuf[buf, base + k, sl])
                    obuf[buf, pp, sl] = acc
            pltpu.sync_copy(obuf.at[buf],
                            out_hbm.at[pl.ds(gbase + s * _CP, _CP)])

        issue(0, 0, semA)

        def step(t, carry):
            s0 = 2 * t
            issue(s0 + 1, 1, semB)
            drain(s0, 0, semA)
            reduce_store(s0, 0)

            @pl.when(s0 + 2 < steps)
            def _():
                issue(s0 + 2, 0, semA)

            drain(s0 + 1, 1, semB)
            reduce_store(s0 + 1, 1)
            return carry

        lax.fori_loop(0, steps // 2, step, 0)

    return pool_kernel


def _run_half(vertices, feature_map, sample_idx):
    bs, n, _ = vertices.shape
    c = feature_map.shape[1]
    r = feature_map.shape[-1]
    p = n // _POOLING_RATE
    d = c * r

    vt = jnp.transpose(vertices, (0, 2, 1))
    vs = jnp.take(vertices, sample_idx, axis=1)
    nbr = _knn_indices(vs, vt)

    fm_t = jnp.transpose(feature_map, (0, 2, 1, 3)).astype(
        jnp.bfloat16).reshape(bs * n, d)
    nbr_flat = jnp.transpose(nbr, (0, 2, 1)).reshape(-1)

    nw = 32
    pw = (bs * p) // nw
    pool = _make_pool_kernel(bs * p, d, pw, bs, p)
    pooled = pool(fm_t, nbr_flat)

    feature_map_pool = pooled.astype(jnp.float32).reshape(
        bs, p, c, r).transpose(0, 2, 1, 3)
    return (vs, feature_map_pool)


def kernel(vertices, feature_map):
    bs, n, _ = vertices.shape
    p = n // _POOLING_RATE
    sample_idx = _sample_indices(n, p)
    nsplit = 2
    hb = bs // nsplit
    parts = [
        _run_half(vertices[h * hb:(h + 1) * hb],
                  feature_map[h * hb:(h + 1) * hb], sample_idx)
        for h in range(nsplit)
    ]
    return (jnp.concatenate([v for v, _ in parts], axis=0),
            jnp.concatenate([f for _, f in parts], axis=0))
